# Initial kernel scaffold; baseline (speedup 1.0000x reference)
import functools

import jax
import jax.numpy as jnp
from jax import lax
from jax.experimental import pallas as pl
from jax.experimental.pallas import tpu as pltpu
from jax.experimental.pallas import tpu_sc as plsc

N = 10000
E = 320000
D = 128
G = 128
D_OUT = 120
ALPHA = 0.5

NC = 2
NS = 16
NW = NC * NS
CH = 80
NCHUNK = E // CH
CPW = NCHUNK // NW
ZR = N // NS
SEG_CHUNKS = N // CH

BN = 1000
_MESH = plsc.VectorSubcoreMesh(core_axis_name="c", subcore_axis_name="s")
_f32 = jnp.float32


def _zero_rows(ref, nrows, ncols):
    @pl.loop(0, nrows)
    def _(i):
        @pl.loop(0, ncols // 16)
        def _(j):
            ref[i, pl.ds(j * 16, 16)] = jnp.zeros((16,), _f32)


@functools.partial(
    pl.kernel,
    out_type=(
        jax.ShapeDtypeStruct((NC, N, 16), _f32),
        jax.ShapeDtypeStruct((NC, N, 16), _f32),
        jax.ShapeDtypeStruct((G, 16), _f32),
    ),
    mesh=_MESH,
    scratch_types=[
        pltpu.VMEM((CH,), jnp.int32),
        pltpu.VMEM((CH,), jnp.int32),
        pltpu.VMEM((ZR, 16), _f32),
        pltpu.VMEM((CH, 16), _f32),
        pltpu.VMEM_SHARED((N, 16), _f32),
        pltpu.VMEM_SHARED((N, 16), _f32),
        pltpu.VMEM_SHARED((G, 16), _f32),
    ],
)
def _sc_degrees(src_hbm, dst_hbm, seg_hbm, din_hbm, dout_hbm, cnt_hbm,
                sidx, didx, zb, ob, acc_i, acc_o, acc_c):
    c = lax.axis_index("c")
    s = lax.axis_index("s")
    base = (c * NS + s) * CPW

    _zero_rows(zb, ZR, 16)

    @pl.loop(0, CH)
    def _(i):
        ob[i, :] = jnp.ones((16,), _f32)

    pltpu.sync_copy(zb, acc_i.at[pl.ds(s * ZR, ZR)])
    pltpu.sync_copy(zb, acc_o.at[pl.ds(s * ZR, ZR)])

    @pl.when(s == 0)
    def _():
        pltpu.sync_copy(zb.at[pl.ds(0, G)], acc_c)

    plsc.subcore_barrier()

    @pl.loop(0, CPW)
    def _(k):
        pltpu.sync_copy(src_hbm.at[base + k], sidx)
        pltpu.sync_copy(dst_hbm.at[base + k], didx)
        pltpu.sync_copy(ob, acc_i.at[didx], add=True)
        pltpu.sync_copy(ob, acc_o.at[sidx], add=True)

    @pl.when(c == 0)
    def _():
        @pl.loop(s, SEG_CHUNKS, step=NS)
        def _(k):
            pltpu.sync_copy(seg_hbm.at[k], sidx)
            pltpu.sync_copy(ob, acc_c.at[sidx], add=True)

    plsc.subcore_barrier()

    pltpu.sync_copy(acc_i.at[pl.ds(s * ZR, ZR)], din_hbm.at[c, pl.ds(s * ZR, ZR)])
    pltpu.sync_copy(acc_o.at[pl.ds(s * ZR, ZR)], dout_hbm.at[c, pl.ds(s * ZR, ZR)])

    @pl.when((c == 0) & (s == 0))
    def _():
        pltpu.sync_copy(acc_c, cnt_hbm)


@functools.partial(
    pl.kernel,
    out_type=(
        jax.ShapeDtypeStruct((NC, N, D), _f32),
        jax.ShapeDtypeStruct((NC, N, D), _f32),
    ),
    mesh=_MESH,
    scratch_types=[
        pltpu.VMEM((CH,), jnp.int32),
        pltpu.VMEM((CH,), jnp.int32),
        pltpu.VMEM((CH, D), _f32),
        pltpu.VMEM((ZR, D), _f32),
        pltpu.VMEM_SHARED((N, D), _f32),
    ],
)
def _sc_spmm(u_hbm, v_hbm, src_hbm, dst_hbm, ain_hbm, aout_hbm,
             gidx, tidx, rows, zb, acc):
    c = lax.axis_index("c")
    s = lax.axis_index("s")
    base = (c * NS + s) * CPW

    _zero_rows(zb, ZR, D)

    def one_pass(tab_hbm, gat_hbm, sct_hbm, out_hbm):
        pltpu.sync_copy(zb, acc.at[pl.ds(s * ZR, ZR)])
        plsc.subcore_barrier()

        @pl.loop(0, CPW)
        def _(k):
            pltpu.sync_copy(gat_hbm.at[base + k], gidx)
            pltpu.sync_copy(tab_hbm.at[gidx], rows)
            pltpu.sync_copy(sct_hbm.at[base + k], tidx)
            pltpu.sync_copy(rows, acc.at[tidx], add=True)

        plsc.subcore_barrier()
        pltpu.sync_copy(acc.at[pl.ds(s * ZR, ZR)], out_hbm.at[c, pl.ds(s * ZR, ZR)])
        plsc.subcore_barrier()

    one_pass(u_hbm, src_hbm, dst_hbm, ain_hbm)
    one_pass(v_hbm, dst_hbm, src_hbm, aout_hbm)


@functools.partial(
    pl.kernel,
    out_type=jax.ShapeDtypeStruct((NC, G, D), _f32),
    mesh=_MESH,
    scratch_types=[
        pltpu.VMEM((CH,), jnp.int32),
        pltpu.VMEM((CH, D), _f32),
        pltpu.VMEM((G // NS, D), _f32),
        pltpu.VMEM_SHARED((G, D), _f32),
    ],
)
def _sc_pool(h_hbm, seg_hbm, sums_hbm, gidx, rows, zb, acc):
    c = lax.axis_index("c")
    s = lax.axis_index("s")
    w = c * NS + s

    _zero_rows(zb, G // NS, D)
    pltpu.sync_copy(zb, acc.at[pl.ds(s * (G // NS), G // NS)])
    plsc.subcore_barrier()

    @pl.loop(w, SEG_CHUNKS, step=NW)
    def _(k):
        pltpu.sync_copy(h_hbm.at[pl.ds(k * CH, CH)], rows)
        pltpu.sync_copy(seg_hbm.at[k], gidx)
        pltpu.sync_copy(rows, acc.at[gidx], add=True)

    plsc.subcore_barrier()

    @pl.when(s == 0)
    def _():
        pltpu.sync_copy(acc, sums_hbm.at[c])


def _dinv(deg):
    return jnp.where(deg > 0, lax.rsqrt(jnp.maximum(deg, 1e-12)), 0.0)


def _tc_prescale(degp_in, degp_out, x):
    def body(dip, dop, x_ref, dvi_ref, dvo_ref, u_ref, v_ref):
        a = dip[...]
        b = dop[...]
        dvi = _dinv((a[0] + a[1])[:, 0:1])
        dvo = _dinv((b[0] + b[1])[:, 0:1])
        xb = x_ref[...]
        dvi_ref[...] = dvi
        dvo_ref[...] = dvo
        u_ref[...] = xb * dvi
        v_ref[...] = xb * dvo

    return pl.pallas_call(
        body,
        grid=(N // BN,),
        in_specs=[
            pl.BlockSpec((NC, BN, 16), lambda i: (0, i, 0)),
            pl.BlockSpec((NC, BN, 16), lambda i: (0, i, 0)),
            pl.BlockSpec((BN, D), lambda i: (i, 0)),
        ],
        out_specs=[
            pl.BlockSpec((BN, 1), lambda i: (i, 0)),
            pl.BlockSpec((BN, 1), lambda i: (i, 0)),
            pl.BlockSpec((BN, D), lambda i: (i, 0)),
            pl.BlockSpec((BN, D), lambda i: (i, 0)),
        ],
        out_shape=(
            jax.ShapeDtypeStruct((N, 1), _f32),
            jax.ShapeDtypeStruct((N, 1), _f32),
            jax.ShapeDtypeStruct((N, D), _f32),
            jax.ShapeDtypeStruct((N, D), _f32),
        ),
    )(degp_in, degp_out, x)


def _tc_layer(ainp, aoutp, dvi, dvo, wi, wo, bi, bo, relu, last):
    def body(ain_ref, aout_ref, dvi_ref, dvo_ref, wi_ref, wo_ref, bi_ref,
             bo_ref, *outs):
        a = ain_ref[...]
        b = aout_ref[...]
        ain = a[0] + a[1]
        aout = b[0] + b[1]
        dvi_b = dvi_ref[...]
        dvo_b = dvo_ref[...]
        xin = dvi_b * jnp.dot(ain, wi_ref[...], preferred_element_type=_f32) + bi_ref[...]
        xout = dvo_b * jnp.dot(aout, wo_ref[...], preferred_element_type=_f32) + bo_ref[...]
        h = ALPHA * xout + (1.0 - ALPHA) * xin
        if relu:
            h = jnp.maximum(h, 0.0)
        if last:
            outs[0][...] = h
        else:
            outs[0][...] = h * dvi_b
            outs[1][...] = h * dvo_b

    n_out = 1 if last else 2
    return pl.pallas_call(
        body,
        grid=(N // BN,),
        in_specs=[
            pl.BlockSpec((NC, BN, D), lambda i: (0, i, 0)),
            pl.BlockSpec((NC, BN, D), lambda i: (0, i, 0)),
            pl.BlockSpec((BN, 1), lambda i: (i, 0)),
            pl.BlockSpec((BN, 1), lambda i: (i, 0)),
            pl.BlockSpec((D, D), lambda i: (0, 0)),
            pl.BlockSpec((D, D), lambda i: (0, 0)),
            pl.BlockSpec((1, D), lambda i: (0, 0)),
            pl.BlockSpec((1, D), lambda i: (0, 0)),
        ],
        out_specs=[pl.BlockSpec((BN, D), lambda i: (i, 0))] * n_out,
        out_shape=tuple(jax.ShapeDtypeStruct((N, D), _f32) for _ in range(n_out)),
    )(ainp, aoutp, dvi, dvo, wi, wo, bi, bo)


def _tc_final(sums_p, counts):
    def body(s_ref, c_ref, o_ref):
        sp = s_ref[...]
        sums = sp[0] + sp[1]
        cnt = c_ref[...][:, 0:1]
        o_ref[...] = (sums / jnp.maximum(cnt, 1.0))[:, :D_OUT]

    return pl.pallas_call(
        body,
        out_shape=jax.ShapeDtypeStruct((G, D_OUT), _f32),
    )(sums_p, counts)


def kernel(x, edge_index, batch_seg, W_in1, b_in1, W_out1, b_out1,
           W_in2, b_in2, W_out2, b_out2, W_in3, b_in3, W_out3, b_out3):
    src2d = edge_index[0].astype(jnp.int32).reshape(NCHUNK, CH)
    dst2d = edge_index[1].astype(jnp.int32).reshape(NCHUNK, CH)
    seg2d = batch_seg.astype(jnp.int32).reshape(SEG_CHUNKS, CH)

    pad = D - D_OUT
    weights = [
        (W_in1, W_out1, b_in1.reshape(1, D), b_out1.reshape(1, D)),
        (W_in2, W_out2, b_in2.reshape(1, D), b_out2.reshape(1, D)),
        (jnp.pad(W_in3, ((0, 0), (0, pad))),
         jnp.pad(W_out3, ((0, 0), (0, pad))),
         jnp.pad(b_in3, (0, pad)).reshape(1, D),
         jnp.pad(b_out3, (0, pad)).reshape(1, D)),
    ]

    degp_in, degp_out, counts = _sc_degrees(src2d, dst2d, seg2d)
    dvi, dvo, u, v = _tc_prescale(degp_in, degp_out, x)

    h3 = None
    for li, (wi, wo, bi, bo) in enumerate(weights):
        last = li == 2
        ainp, aoutp = _sc_spmm(u, v, src2d, dst2d)
        outs = _tc_layer(ainp, aoutp, dvi, dvo, wi, wo, bi, bo,
                         relu=not last, last=last)
        if last:
            h3 = outs[0] if isinstance(outs, (tuple, list)) else outs
        else:
            u, v = outs

    sums_p = _sc_pool(h3, seg2d)
    return _tc_final(sums_p, counts)


# speedup vs baseline: 6.5506x; 6.5506x over previous
import functools

import jax
import jax.numpy as jnp
from jax import lax
from jax.experimental import pallas as pl
from jax.experimental.pallas import tpu as pltpu
from jax.experimental.pallas import tpu_sc as plsc

N = 10000
NP = 10240
E = 320000
D = 128
G = 128
D_OUT = 120
ALPHA = 0.5

NC = 2
NS = 16
NW = NC * NS
CH = 80
NCHUNK = E // CH
CPW = NCHUNK // NW
ZR = NP // NS
SEG_CHUNKS = N // CH

BN = 640
_f32 = jnp.float32


@functools.cache
def _mesh():
    return plsc.VectorSubcoreMesh(core_axis_name="c", subcore_axis_name="s",
                                  num_cores=NC, num_subcores=NS)


@functools.cache
def _sc_spmm_kernel():
    return pl.kernel(
        _sc_spmm_body,
        out_type=(
            jax.ShapeDtypeStruct((NC, NP, D), _f32),
            jax.ShapeDtypeStruct((NC, NP, D), _f32),
        ),
        mesh=_mesh(),
        scratch_types=[
            pltpu.VMEM((1, CH), jnp.int32),
            pltpu.VMEM((1, CH), jnp.int32),
            pltpu.VMEM((CH, D), _f32),
            pltpu.VMEM_SHARED((NP, D), _f32),
        ],
    )


def _sc_spmm_body(u_hbm, v_hbm, src_hbm, dst_hbm, z_hbm, ain_hbm, aout_hbm,
                  gidx, tidx, rows, acc):
    c = lax.axis_index("c")
    s = lax.axis_index("s")
    base = (c * NS + s) * CPW

    def one_pass(tab_hbm, gat_hbm, sct_hbm, out_hbm):
        pltpu.sync_copy(z_hbm, acc.at[pl.ds(s * ZR, ZR)])
        plsc.subcore_barrier()

        @pl.loop(0, CPW)
        def _(k):
            pltpu.sync_copy(gat_hbm.at[base + k], gidx)
            pltpu.sync_copy(tab_hbm.at[gidx.at[0]], rows)
            pltpu.sync_copy(sct_hbm.at[base + k], tidx)
            pltpu.sync_copy(rows, acc.at[tidx.at[0]], add=True)

        plsc.subcore_barrier()
        pltpu.sync_copy(acc.at[pl.ds(s * ZR, ZR)], out_hbm.at[c, pl.ds(s * ZR, ZR)])
        plsc.subcore_barrier()

    one_pass(u_hbm, src_hbm, dst_hbm, ain_hbm)
    one_pass(v_hbm, dst_hbm, src_hbm, aout_hbm)


@functools.cache
def _sc_pool_kernel():
    return pl.kernel(
        _sc_pool_body,
        out_type=jax.ShapeDtypeStruct((NC, G, D), _f32),
        mesh=_mesh(),
        scratch_types=[
            pltpu.VMEM((1, CH), jnp.int32),
            pltpu.VMEM((CH, D), _f32),
            pltpu.VMEM_SHARED((G, D), _f32),
        ],
    )


def _sc_pool_body(h_hbm, seg_hbm, z_hbm, sums_hbm, gidx, rows, acc):
    c = lax.axis_index("c")
    s = lax.axis_index("s")
    w = c * NS + s
    gr = G // NS

    pltpu.sync_copy(z_hbm.at[pl.ds(0, gr)], acc.at[pl.ds(s * gr, gr)])
    plsc.subcore_barrier()

    @pl.loop(w, SEG_CHUNKS, step=NW)
    def _(k):
        pltpu.sync_copy(h_hbm.at[pl.ds(k * CH, CH)], rows)
        pltpu.sync_copy(seg_hbm.at[k], gidx)
        pltpu.sync_copy(rows, acc.at[gidx.at[0]], add=True)

    plsc.subcore_barrier()

    @pl.when(s == 0)
    def _():
        pltpu.sync_copy(acc, sums_hbm.at[c])


def _dinv(deg):
    return jnp.where(deg > 0, lax.rsqrt(jnp.maximum(deg, 1e-12)), 0.0)


def _tc_prescale(degp_in, degp_out, x):
    def body(dip, dop, x_ref, dvi_ref, dvo_ref, u_ref, v_ref):
        a = dip[...]
        b = dop[...]
        dvi = _dinv((a[0] + a[1])[:, 0:1])
        dvo = _dinv((b[0] + b[1])[:, 0:1])
        xb = x_ref[...]
        dvi_ref[...] = dvi
        dvo_ref[...] = dvo
        u_ref[...] = xb * dvi
        v_ref[...] = xb * dvo

    return pl.pallas_call(
        body,
        grid=(NP // BN,),
        in_specs=[
            pl.BlockSpec((NC, BN, D), lambda i: (0, i, 0)),
            pl.BlockSpec((NC, BN, D), lambda i: (0, i, 0)),
            pl.BlockSpec((BN, D), lambda i: (i, 0)),
        ],
        out_specs=[
            pl.BlockSpec((BN, 1), lambda i: (i, 0)),
            pl.BlockSpec((BN, 1), lambda i: (i, 0)),
            pl.BlockSpec((BN, D), lambda i: (i, 0)),
            pl.BlockSpec((BN, D), lambda i: (i, 0)),
        ],
        out_shape=(
            jax.ShapeDtypeStruct((NP, 1), _f32),
            jax.ShapeDtypeStruct((NP, 1), _f32),
            jax.ShapeDtypeStruct((NP, D), _f32),
            jax.ShapeDtypeStruct((NP, D), _f32),
        ),
    )(degp_in, degp_out, x)


def _tc_layer(ainp, aoutp, dvi, dvo, wi, wo, bi, bo, relu, last):
    def body(ain_ref, aout_ref, dvi_ref, dvo_ref, wi_ref, wo_ref, bi_ref,
             bo_ref, *outs):
        a = ain_ref[...]
        b = aout_ref[...]
        ain = a[0] + a[1]
        aout = b[0] + b[1]
        dvi_b = dvi_ref[...]
        dvo_b = dvo_ref[...]
        xin = dvi_b * jnp.dot(ain, wi_ref[...], preferred_element_type=_f32) + bi_ref[...]
        xout = dvo_b * jnp.dot(aout, wo_ref[...], preferred_element_type=_f32) + bo_ref[...]
        h = ALPHA * xout + (1.0 - ALPHA) * xin
        if relu:
            h = jnp.maximum(h, 0.0)
        if last:
            outs[0][...] = h
        else:
            outs[0][...] = h * dvi_b
            outs[1][...] = h * dvo_b

    n_out = 1 if last else 2
    return pl.pallas_call(
        body,
        grid=(NP // BN,),
        in_specs=[
            pl.BlockSpec((NC, BN, D), lambda i: (0, i, 0)),
            pl.BlockSpec((NC, BN, D), lambda i: (0, i, 0)),
            pl.BlockSpec((BN, 1), lambda i: (i, 0)),
            pl.BlockSpec((BN, 1), lambda i: (i, 0)),
            pl.BlockSpec((D, D), lambda i: (0, 0)),
            pl.BlockSpec((D, D), lambda i: (0, 0)),
            pl.BlockSpec((1, D), lambda i: (0, 0)),
            pl.BlockSpec((1, D), lambda i: (0, 0)),
        ],
        out_specs=[pl.BlockSpec((BN, D), lambda i: (i, 0))] * n_out,
        out_shape=tuple(jax.ShapeDtypeStruct((NP, D), _f32) for _ in range(n_out)),
    )(ainp, aoutp, dvi, dvo, wi, wo, bi, bo)


def _tc_final(sums_p, counts_p):
    def body(s_ref, c_ref, o_ref):
        sp = s_ref[...]
        cp = c_ref[...]
        sums = sp[0] + sp[1]
        cnt = (cp[0] + cp[1])[:, 0:1]
        o_ref[...] = (sums / jnp.maximum(cnt, 1.0))[:, :D_OUT]

    return pl.pallas_call(
        body,
        out_shape=jax.ShapeDtypeStruct((G, D_OUT), _f32),
    )(sums_p, counts_p)


def kernel(x, edge_index, batch_seg, W_in1, b_in1, W_out1, b_out1,
           W_in2, b_in2, W_out2, b_out2, W_in3, b_in3, W_out3, b_out3):
    src3d = edge_index[0].astype(jnp.int32).reshape(NCHUNK, 1, CH)
    dst3d = edge_index[1].astype(jnp.int32).reshape(NCHUNK, 1, CH)
    seg3d = batch_seg.astype(jnp.int32).reshape(SEG_CHUNKS, 1, CH)
    x_p = jnp.pad(x, ((0, NP - N), (0, 0)))
    ones_np = jnp.ones((NP, D), _f32)
    zrow = jnp.zeros((ZR, D), _f32)

    pad = D - D_OUT
    weights = [
        (W_in1, W_out1, b_in1.reshape(1, D), b_out1.reshape(1, D)),
        (W_in2, W_out2, b_in2.reshape(1, D), b_out2.reshape(1, D)),
        (jnp.pad(W_in3, ((0, 0), (0, pad))),
         jnp.pad(W_out3, ((0, 0), (0, pad))),
         jnp.pad(b_in3, (0, pad)).reshape(1, D),
         jnp.pad(b_out3, (0, pad)).reshape(1, D)),
    ]

    degp_in, degp_out = _sc_spmm_kernel()(ones_np, ones_np, src3d, dst3d, zrow)
    counts_p = _sc_pool_kernel()(ones_np, seg3d, zrow)
    dvi, dvo, u, v = _tc_prescale(degp_in, degp_out, x_p)

    h3 = None
    for li, (wi, wo, bi, bo) in enumerate(weights):
        last = li == 2
        ainp, aoutp = _sc_spmm_kernel()(u, v, src3d, dst3d, zrow)
        outs = _tc_layer(ainp, aoutp, dvi, dvo, wi, wo, bi, bo,
                         relu=not last, last=last)
        if last:
            h3 = outs[0] if isinstance(outs, (tuple, list)) else outs
        else:
            u, v = outs

    sums_p = _sc_pool_kernel()(h3, seg3d, zrow)
    return _tc_final(sums_p, counts_p)


# speedup vs baseline: 12.5478x; 1.9155x over previous
import functools

import jax
import jax.numpy as jnp
from jax import lax
from jax.experimental import pallas as pl
from jax.experimental.pallas import tpu as pltpu
from jax.experimental.pallas import tpu_sc as plsc

N = 10000
NP = 10240
E = 320000
D = 128
G = 128
D_OUT = 120
ALPHA = 0.5

NC = 2
NS = 16
NW = NC * NS
CH = 80
NCHUNK = E // CH
CPW = NCHUNK // NW
ZR = NP // NS
SEG_CHUNKS = N // CH
IB = 64
_BLOCKS = ((0, 64), (64, 61))

BN = 640
_f32 = jnp.float32


@functools.cache
def _mesh():
    return plsc.VectorSubcoreMesh(core_axis_name="c", subcore_axis_name="s",
                                  num_cores=NC, num_subcores=NS)


@functools.cache
def _sc_spmm_kernel():
    return pl.kernel(
        _sc_spmm_body,
        out_type=(
            jax.ShapeDtypeStruct((NC, NP, D), _f32),
            jax.ShapeDtypeStruct((NC, NP, D), _f32),
        ),
        mesh=_mesh(),
        scratch_types=[
            pltpu.VMEM((IB, 1, CH), jnp.int32),
            pltpu.VMEM((IB, 1, CH), jnp.int32),
            pltpu.VMEM((CH, D), _f32),
            pltpu.VMEM((CH, D), _f32),
            pltpu.VMEM_SHARED((NP, D), _f32),
            pltpu.SemaphoreType.DMA,
        ],
    )


def _sc_spmm_body(u_hbm, v_hbm, src_hbm, dst_hbm, z_hbm, ain_hbm, aout_hbm,
                  gib, tib, r0, r1, acc, gsem):
    c = lax.axis_index("c")
    s = lax.axis_index("s")
    base = (c * NS + s) * CPW

    def one_pass(tab_hbm, gat_hbm, sct_hbm, out_hbm):
        pltpu.sync_copy(z_hbm, acc.at[pl.ds(s * ZR, ZR)])
        plsc.subcore_barrier()

        def gather(k, rbuf):
            pltpu.async_copy(tab_hbm.at[gib.at[k, 0]], rbuf, gsem)

        def wait_gather(rbuf):
            pltpu.make_async_copy(tab_hbm.at[gib.at[0, 0]], rbuf, gsem).wait()

        def scatter(k, rbuf):
            pltpu.sync_copy(rbuf, acc.at[tib.at[k, 0]], add=True)

        for bs, bn in _BLOCKS:
            pltpu.sync_copy(gat_hbm.at[pl.ds(base + bs, bn)], gib.at[pl.ds(0, bn)])
            pltpu.sync_copy(sct_hbm.at[pl.ds(base + bs, bn)], tib.at[pl.ds(0, bn)])
            gather(0, r0)

            @pl.loop(0, bn // 2)
            def _(i):
                k = i * 2
                wait_gather(r0)
                gather(k + 1, r1)
                scatter(k, r0)
                wait_gather(r1)

                @pl.when(k + 2 < bn)
                def _():
                    gather(k + 2, r0)

                scatter(k + 1, r1)

            if bn % 2:
                wait_gather(r0)
                scatter(bn - 1, r0)

        plsc.subcore_barrier()
        pltpu.sync_copy(acc.at[pl.ds(s * ZR, ZR)], out_hbm.at[c, pl.ds(s * ZR, ZR)])
        plsc.subcore_barrier()

    one_pass(u_hbm, src_hbm, dst_hbm, ain_hbm)
    one_pass(v_hbm, dst_hbm, src_hbm, aout_hbm)


@functools.cache
def _sc_pool_kernel():
    return pl.kernel(
        _sc_pool_body,
        out_type=jax.ShapeDtypeStruct((NC, G, D), _f32),
        mesh=_mesh(),
        scratch_types=[
            pltpu.VMEM((1, CH), jnp.int32),
            pltpu.VMEM((CH, D), _f32),
            pltpu.VMEM_SHARED((G, D), _f32),
        ],
    )


def _sc_pool_body(h_hbm, seg_hbm, z_hbm, sums_hbm, gidx, rows, acc):
    c = lax.axis_index("c")
    s = lax.axis_index("s")
    w = c * NS + s
    gr = G // NS

    pltpu.sync_copy(z_hbm.at[pl.ds(0, gr)], acc.at[pl.ds(s * gr, gr)])
    plsc.subcore_barrier()

    @pl.loop(w, SEG_CHUNKS, step=NW)
    def _(k):
        pltpu.sync_copy(h_hbm.at[pl.ds(k * CH, CH)], rows)
        pltpu.sync_copy(seg_hbm.at[k], gidx)
        pltpu.sync_copy(rows, acc.at[gidx.at[0]], add=True)

    plsc.subcore_barrier()

    @pl.when(s == 0)
    def _():
        pltpu.sync_copy(acc, sums_hbm.at[c])


def _dinv(deg):
    return jnp.where(deg > 0, lax.rsqrt(jnp.maximum(deg, 1e-12)), 0.0)


def _tc_prescale(degp_in, degp_out, x):
    def body(dip, dop, x_ref, dvi_ref, dvo_ref, u_ref, v_ref):
        a = dip[...]
        b = dop[...]
        dvi = _dinv((a[0] + a[1])[:, 0:1])
        dvo = _dinv((b[0] + b[1])[:, 0:1])
        xb = x_ref[...]
        dvi_ref[...] = dvi
        dvo_ref[...] = dvo
        u_ref[...] = xb * dvi
        v_ref[...] = xb * dvo

    return pl.pallas_call(
        body,
        grid=(NP // BN,),
        in_specs=[
            pl.BlockSpec((NC, BN, D), lambda i: (0, i, 0)),
            pl.BlockSpec((NC, BN, D), lambda i: (0, i, 0)),
            pl.BlockSpec((BN, D), lambda i: (i, 0)),
        ],
        out_specs=[
            pl.BlockSpec((BN, 1), lambda i: (i, 0)),
            pl.BlockSpec((BN, 1), lambda i: (i, 0)),
            pl.BlockSpec((BN, D), lambda i: (i, 0)),
            pl.BlockSpec((BN, D), lambda i: (i, 0)),
        ],
        out_shape=(
            jax.ShapeDtypeStruct((NP, 1), _f32),
            jax.ShapeDtypeStruct((NP, 1), _f32),
            jax.ShapeDtypeStruct((NP, D), _f32),
            jax.ShapeDtypeStruct((NP, D), _f32),
        ),
    )(degp_in, degp_out, x)


def _tc_layer(ainp, aoutp, dvi, dvo, wi, wo, bi, bo, relu, last):
    def body(ain_ref, aout_ref, dvi_ref, dvo_ref, wi_ref, wo_ref, bi_ref,
             bo_ref, *outs):
        a = ain_ref[...]
        b = aout_ref[...]
        ain = a[0] + a[1]
        aout = b[0] + b[1]
        dvi_b = dvi_ref[...]
        dvo_b = dvo_ref[...]
        xin = dvi_b * jnp.dot(ain, wi_ref[...], preferred_element_type=_f32) + bi_ref[...]
        xout = dvo_b * jnp.dot(aout, wo_ref[...], preferred_element_type=_f32) + bo_ref[...]
        h = ALPHA * xout + (1.0 - ALPHA) * xin
        if relu:
            h = jnp.maximum(h, 0.0)
        if last:
            outs[0][...] = h
        else:
            outs[0][...] = h * dvi_b
            outs[1][...] = h * dvo_b

    n_out = 1 if last else 2
    return pl.pallas_call(
        body,
        grid=(NP // BN,),
        in_specs=[
            pl.BlockSpec((NC, BN, D), lambda i: (0, i, 0)),
            pl.BlockSpec((NC, BN, D), lambda i: (0, i, 0)),
            pl.BlockSpec((BN, 1), lambda i: (i, 0)),
            pl.BlockSpec((BN, 1), lambda i: (i, 0)),
            pl.BlockSpec((D, D), lambda i: (0, 0)),
            pl.BlockSpec((D, D), lambda i: (0, 0)),
            pl.BlockSpec((1, D), lambda i: (0, 0)),
            pl.BlockSpec((1, D), lambda i: (0, 0)),
        ],
        out_specs=[pl.BlockSpec((BN, D), lambda i: (i, 0))] * n_out,
        out_shape=tuple(jax.ShapeDtypeStruct((NP, D), _f32) for _ in range(n_out)),
    )(ainp, aoutp, dvi, dvo, wi, wo, bi, bo)


def _tc_final(sums_p, counts_p):
    def body(s_ref, c_ref, o_ref):
        sp = s_ref[...]
        cp = c_ref[...]
        sums = sp[0] + sp[1]
        cnt = (cp[0] + cp[1])[:, 0:1]
        o_ref[...] = (sums / jnp.maximum(cnt, 1.0))[:, :D_OUT]

    return pl.pallas_call(
        body,
        out_shape=jax.ShapeDtypeStruct((G, D_OUT), _f32),
    )(sums_p, counts_p)


def kernel(x, edge_index, batch_seg, W_in1, b_in1, W_out1, b_out1,
           W_in2, b_in2, W_out2, b_out2, W_in3, b_in3, W_out3, b_out3):
    src3d = edge_index[0].astype(jnp.int32).reshape(NCHUNK, 1, CH)
    dst3d = edge_index[1].astype(jnp.int32).reshape(NCHUNK, 1, CH)
    seg3d = batch_seg.astype(jnp.int32).reshape(SEG_CHUNKS, 1, CH)
    x_p = jnp.pad(x, ((0, NP - N), (0, 0)))
    ones_np = jnp.ones((NP, D), _f32)
    zrow = jnp.zeros((ZR, D), _f32)

    pad = D - D_OUT
    weights = [
        (W_in1, W_out1, b_in1.reshape(1, D), b_out1.reshape(1, D)),
        (W_in2, W_out2, b_in2.reshape(1, D), b_out2.reshape(1, D)),
        (jnp.pad(W_in3, ((0, 0), (0, pad))),
         jnp.pad(W_out3, ((0, 0), (0, pad))),
         jnp.pad(b_in3, (0, pad)).reshape(1, D),
         jnp.pad(b_out3, (0, pad)).reshape(1, D)),
    ]

    degp_in, degp_out = _sc_spmm_kernel()(ones_np, ones_np, src3d, dst3d, zrow)
    counts_p = _sc_pool_kernel()(ones_np, seg3d, zrow)
    dvi, dvo, u, v = _tc_prescale(degp_in, degp_out, x_p)

    h3 = None
    for li, (wi, wo, bi, bo) in enumerate(weights):
        last = li == 2
        ainp, aoutp = _sc_spmm_kernel()(u, v, src3d, dst3d, zrow)
        outs = _tc_layer(ainp, aoutp, dvi, dvo, wi, wo, bi, bo,
                         relu=not last, last=last)
        if last:
            h3 = outs[0] if isinstance(outs, (tuple, list)) else outs
        else:
            u, v = outs

    sums_p = _sc_pool_kernel()(h3, seg3d, zrow)
    return _tc_final(sums_p, counts_p)


# speedup vs baseline: 14.2305x; 1.1341x over previous
import functools

import jax
import jax.numpy as jnp
from jax import lax
from jax.experimental import pallas as pl
from jax.experimental.pallas import tpu as pltpu
from jax.experimental.pallas import tpu_sc as plsc

N = 10000
NP = 10240
E = 320000
D = 128
G = 128
D_OUT = 120
ALPHA = 0.5

NC = 2
NS = 16
NW = NC * NS
CH = 80
NCHUNK = E // CH
CPW = NCHUNK // NW
ZR = NP // NS
SEG_CHUNKS = N // CH
IB = 64
_BLOCKS = ((0, 64), (64, 61))

BN = 640
_f32 = jnp.float32


@functools.cache
def _mesh():
    return plsc.VectorSubcoreMesh(core_axis_name="c", subcore_axis_name="s",
                                  num_cores=NC, num_subcores=NS)


@functools.cache
def _sc_spmm_kernel():
    return pl.kernel(
        _sc_spmm_body,
        out_type=(
            jax.ShapeDtypeStruct((NC, NP, D), _f32),
            jax.ShapeDtypeStruct((NC, NP, D), _f32),
        ),
        mesh=_mesh(),
        scratch_types=[
            pltpu.VMEM((IB, 1, CH), jnp.int32),
            pltpu.VMEM((IB, 1, CH), jnp.int32),
            pltpu.VMEM((CH, D), _f32),
            pltpu.VMEM((CH, D), _f32),
            pltpu.VMEM_SHARED((NP, D), _f32),
            pltpu.SemaphoreType.DMA,
            pltpu.SemaphoreType.DMA,
            pltpu.SemaphoreType.DMA,
        ],
    )


def _sc_spmm_body(u_hbm, v_hbm, src_hbm, dst_hbm, z_hbm, ain_hbm, aout_hbm,
                  gib, tib, r0, r1, acc, gsem, ssem0, ssem1):
    c = lax.axis_index("c")
    s = lax.axis_index("s")
    base = (c * NS + s) * CPW

    def one_pass(tab_hbm, gat_hbm, sct_hbm, out_hbm):
        pltpu.sync_copy(z_hbm, acc.at[pl.ds(s * ZR, ZR)])
        plsc.subcore_barrier()

        def gather(k, rbuf):
            pltpu.async_copy(tab_hbm.at[gib.at[k, 0]], rbuf, gsem)

        def wait_gather(rbuf):
            pltpu.make_async_copy(tab_hbm.at[gib.at[0, 0]], rbuf, gsem).wait()

        def scatter(k, rbuf, sem):
            pltpu.async_copy(rbuf, acc.at[tib.at[k, 0]], sem, add=True)

        def wait_scatter(rbuf, sem):
            pltpu.make_async_copy(rbuf, acc.at[tib.at[0, 0]], sem).wait()

        for bs, bn in _BLOCKS:
            pltpu.sync_copy(gat_hbm.at[pl.ds(base + bs, bn)], gib.at[pl.ds(0, bn)])
            pltpu.sync_copy(sct_hbm.at[pl.ds(base + bs, bn)], tib.at[pl.ds(0, bn)])
            gather(0, r0)

            @pl.loop(0, bn // 2)
            def _(i):
                k = i * 2
                wait_gather(r0)

                @pl.when(i > 0)
                def _():
                    wait_scatter(r1, ssem1)

                gather(k + 1, r1)
                scatter(k, r0, ssem0)
                wait_gather(r1)

                @pl.when(k + 2 < bn)
                def _():
                    wait_scatter(r0, ssem0)
                    gather(k + 2, r0)

                scatter(k + 1, r1, ssem1)

            if bn % 2:
                wait_gather(r0)
                wait_scatter(r1, ssem1)
                pltpu.sync_copy(r0, acc.at[tib.at[bn - 1, 0]], add=True)
            else:
                wait_scatter(r0, ssem0)
                wait_scatter(r1, ssem1)

        plsc.subcore_barrier()
        pltpu.sync_copy(acc.at[pl.ds(s * ZR, ZR)], out_hbm.at[c, pl.ds(s * ZR, ZR)])
        plsc.subcore_barrier()

    one_pass(u_hbm, src_hbm, dst_hbm, ain_hbm)
    one_pass(v_hbm, dst_hbm, src_hbm, aout_hbm)


@functools.cache
def _sc_deg_kernel():
    return pl.kernel(
        _sc_deg_body,
        out_type=(
            jax.ShapeDtypeStruct((NC, NP, D), _f32),
            jax.ShapeDtypeStruct((NC, NP, D), _f32),
        ),
        mesh=_mesh(),
        scratch_types=[
            pltpu.VMEM((IB, 1, CH), jnp.int32),
            pltpu.VMEM((CH, D), _f32),
            pltpu.VMEM_SHARED((NP, D), _f32),
        ],
    )


def _sc_deg_body(src_hbm, dst_hbm, z_hbm, ones_hbm, din_hbm, dout_hbm,
                 tib, ob, acc):
    c = lax.axis_index("c")
    s = lax.axis_index("s")
    base = (c * NS + s) * CPW

    pltpu.sync_copy(ones_hbm.at[pl.ds(0, CH)], ob)

    def one_pass(sct_hbm, out_hbm):
        pltpu.sync_copy(z_hbm, acc.at[pl.ds(s * ZR, ZR)])
        plsc.subcore_barrier()

        for bs, bn in _BLOCKS:
            pltpu.sync_copy(sct_hbm.at[pl.ds(base + bs, bn)], tib.at[pl.ds(0, bn)])

            @pl.loop(0, bn)
            def _(k):
                pltpu.sync_copy(ob, acc.at[tib.at[k, 0]], add=True)

        plsc.subcore_barrier()
        pltpu.sync_copy(acc.at[pl.ds(s * ZR, ZR)], out_hbm.at[c, pl.ds(s * ZR, ZR)])
        plsc.subcore_barrier()

    one_pass(dst_hbm, din_hbm)
    one_pass(src_hbm, dout_hbm)


@functools.cache
def _sc_pool_kernel():
    return pl.kernel(
        _sc_pool_body,
        out_type=jax.ShapeDtypeStruct((NC, G, D), _f32),
        mesh=_mesh(),
        scratch_types=[
            pltpu.VMEM((1, CH), jnp.int32),
            pltpu.VMEM((CH, D), _f32),
            pltpu.VMEM_SHARED((G, D), _f32),
        ],
    )


def _sc_pool_body(h_hbm, seg_hbm, z_hbm, sums_hbm, gidx, rows, acc):
    c = lax.axis_index("c")
    s = lax.axis_index("s")
    w = c * NS + s
    gr = G // NS

    pltpu.sync_copy(z_hbm.at[pl.ds(0, gr)], acc.at[pl.ds(s * gr, gr)])
    plsc.subcore_barrier()

    @pl.loop(w, SEG_CHUNKS, step=NW)
    def _(k):
        pltpu.sync_copy(h_hbm.at[pl.ds(k * CH, CH)], rows)
        pltpu.sync_copy(seg_hbm.at[k], gidx)
        pltpu.sync_copy(rows, acc.at[gidx.at[0]], add=True)

    plsc.subcore_barrier()

    @pl.when(s == 0)
    def _():
        pltpu.sync_copy(acc, sums_hbm.at[c])


def _dinv(deg):
    return jnp.where(deg > 0, lax.rsqrt(jnp.maximum(deg, 1e-12)), 0.0)


def _tc_prescale(degp_in, degp_out, x):
    def body(dip, dop, x_ref, dvi_ref, dvo_ref, u_ref, v_ref):
        a = dip[...]
        b = dop[...]
        dvi = _dinv((a[0] + a[1])[:, 0:1])
        dvo = _dinv((b[0] + b[1])[:, 0:1])
        xb = x_ref[...]
        dvi_ref[...] = dvi
        dvo_ref[...] = dvo
        u_ref[...] = xb * dvi
        v_ref[...] = xb * dvo

    return pl.pallas_call(
        body,
        grid=(NP // BN,),
        in_specs=[
            pl.BlockSpec((NC, BN, D), lambda i: (0, i, 0)),
            pl.BlockSpec((NC, BN, D), lambda i: (0, i, 0)),
            pl.BlockSpec((BN, D), lambda i: (i, 0)),
        ],
        out_specs=[
            pl.BlockSpec((BN, 1), lambda i: (i, 0)),
            pl.BlockSpec((BN, 1), lambda i: (i, 0)),
            pl.BlockSpec((BN, D), lambda i: (i, 0)),
            pl.BlockSpec((BN, D), lambda i: (i, 0)),
        ],
        out_shape=(
            jax.ShapeDtypeStruct((NP, 1), _f32),
            jax.ShapeDtypeStruct((NP, 1), _f32),
            jax.ShapeDtypeStruct((NP, D), _f32),
            jax.ShapeDtypeStruct((NP, D), _f32),
        ),
    )(degp_in, degp_out, x)


def _tc_layer(ainp, aoutp, dvi, dvo, wi, wo, bi, bo, relu, last):
    def body(ain_ref, aout_ref, dvi_ref, dvo_ref, wi_ref, wo_ref, bi_ref,
             bo_ref, *outs):
        a = ain_ref[...]
        b = aout_ref[...]
        ain = a[0] + a[1]
        aout = b[0] + b[1]
        dvi_b = dvi_ref[...]
        dvo_b = dvo_ref[...]
        xin = dvi_b * jnp.dot(ain, wi_ref[...], preferred_element_type=_f32) + bi_ref[...]
        xout = dvo_b * jnp.dot(aout, wo_ref[...], preferred_element_type=_f32) + bo_ref[...]
        h = ALPHA * xout + (1.0 - ALPHA) * xin
        if relu:
            h = jnp.maximum(h, 0.0)
        if last:
            outs[0][...] = h
        else:
            outs[0][...] = h * dvi_b
            outs[1][...] = h * dvo_b

    n_out = 1 if last else 2
    return pl.pallas_call(
        body,
        grid=(NP // BN,),
        in_specs=[
            pl.BlockSpec((NC, BN, D), lambda i: (0, i, 0)),
            pl.BlockSpec((NC, BN, D), lambda i: (0, i, 0)),
            pl.BlockSpec((BN, 1), lambda i: (i, 0)),
            pl.BlockSpec((BN, 1), lambda i: (i, 0)),
            pl.BlockSpec((D, D), lambda i: (0, 0)),
            pl.BlockSpec((D, D), lambda i: (0, 0)),
            pl.BlockSpec((1, D), lambda i: (0, 0)),
            pl.BlockSpec((1, D), lambda i: (0, 0)),
        ],
        out_specs=[pl.BlockSpec((BN, D), lambda i: (i, 0))] * n_out,
        out_shape=tuple(jax.ShapeDtypeStruct((NP, D), _f32) for _ in range(n_out)),
    )(ainp, aoutp, dvi, dvo, wi, wo, bi, bo)


def _tc_final(sums_p, counts_p):
    def body(s_ref, c_ref, o_ref):
        sp = s_ref[...]
        cp = c_ref[...]
        sums = sp[0] + sp[1]
        cnt = (cp[0] + cp[1])[:, 0:1]
        o_ref[...] = (sums / jnp.maximum(cnt, 1.0))[:, :D_OUT]

    return pl.pallas_call(
        body,
        out_shape=jax.ShapeDtypeStruct((G, D_OUT), _f32),
    )(sums_p, counts_p)


def kernel(x, edge_index, batch_seg, W_in1, b_in1, W_out1, b_out1,
           W_in2, b_in2, W_out2, b_out2, W_in3, b_in3, W_out3, b_out3):
    src3d = edge_index[0].astype(jnp.int32).reshape(NCHUNK, 1, CH)
    dst3d = edge_index[1].astype(jnp.int32).reshape(NCHUNK, 1, CH)
    seg3d = batch_seg.astype(jnp.int32).reshape(SEG_CHUNKS, 1, CH)
    x_p = jnp.pad(x, ((0, NP - N), (0, 0)))
    ones_np = jnp.ones((NP, D), _f32)
    zrow = jnp.zeros((ZR, D), _f32)

    pad = D - D_OUT
    weights = [
        (W_in1, W_out1, b_in1.reshape(1, D), b_out1.reshape(1, D)),
        (W_in2, W_out2, b_in2.reshape(1, D), b_out2.reshape(1, D)),
        (jnp.pad(W_in3, ((0, 0), (0, pad))),
         jnp.pad(W_out3, ((0, 0), (0, pad))),
         jnp.pad(b_in3, (0, pad)).reshape(1, D),
         jnp.pad(b_out3, (0, pad)).reshape(1, D)),
    ]

    degp_in, degp_out = _sc_deg_kernel()(src3d, dst3d, zrow, ones_np)
    counts_p = _sc_pool_kernel()(ones_np, seg3d, zrow)
    dvi, dvo, u, v = _tc_prescale(degp_in, degp_out, x_p)

    h3 = None
    for li, (wi, wo, bi, bo) in enumerate(weights):
        last = li == 2
        ainp, aoutp = _sc_spmm_kernel()(u, v, src3d, dst3d, zrow)
        outs = _tc_layer(ainp, aoutp, dvi, dvo, wi, wo, bi, bo,
                         relu=not last, last=last)
        if last:
            h3 = outs[0] if isinstance(outs, (tuple, list)) else outs
        else:
            u, v = outs

    sums_p = _sc_pool_kernel()(h3, seg3d, zrow)
    return _tc_final(sums_p, counts_p)
